# baseline (speedup 1.0000x reference)
import numpy as np

import jax
import jax.numpy as jnp
from jax import lax
from jax.experimental import pallas as pl
from jax.experimental.pallas import tpu as pltpu
from jax.experimental.pallas import tpu_sc as plsc

_B = 1024
_L = 50
_E = 50
_D = 128
_BL = _B * _L

_NC = 2
_NS = 16
_NW = _NC * _NS
_PER_W = _BL // _NW
_CH = 80
_NCH = _PER_W // _CH

_G = 4
_S = 64
_R = _G * _S
_GL = _G * _L
_NSTACK = 8
_NEG = -9e15


def _sc_gather_body(emb_hbm, emb2_hbm, idx_hbm, out1_hbm, out2_hbm,
                    idx_v, b1a, b2a, b1b, b2b, s1a, s2a, s1b, s2b):
    wid = lax.axis_index("s") * _NC + lax.axis_index("c")
    pltpu.sync_copy(idx_hbm.at[wid], idx_v)
    base = wid * _PER_W

    def pair(c, carry):
        ca = 2 * c
        off_a = base + ca * _CH
        off_b = off_a + _CH
        cp1a = pltpu.async_copy(emb_hbm.at[idx_v.at[ca]], b1a, s1a)
        cp2a = pltpu.async_copy(emb2_hbm.at[idx_v.at[ca]], b2a, s2a)
        cp1b = pltpu.async_copy(emb_hbm.at[idx_v.at[ca + 1]], b1b, s1b)
        cp2b = pltpu.async_copy(emb2_hbm.at[idx_v.at[ca + 1]], b2b, s2b)
        cp1a.wait()
        pltpu.sync_copy(b1a, out1_hbm.at[pl.ds(off_a, _CH)])
        cp2a.wait()
        pltpu.sync_copy(b2a, out2_hbm.at[pl.ds(off_a, _CH)])
        cp1b.wait()
        pltpu.sync_copy(b1b, out1_hbm.at[pl.ds(off_b, _CH)])
        cp2b.wait()
        pltpu.sync_copy(b2b, out2_hbm.at[pl.ds(off_b, _CH)])
        return carry

    lax.fori_loop(0, _NCH // 2, pair, 0)


def _sc_gather2(emb, emb2, idx3):
    mesh = plsc.VectorSubcoreMesh(core_axis_name="c", subcore_axis_name="s")
    fn = pl.kernel(
        _sc_gather_body,
        out_type=(
            jax.ShapeDtypeStruct((_BL, _D), jnp.float32),
            jax.ShapeDtypeStruct((_BL, _D), jnp.float32),
        ),
        mesh=mesh,
        scratch_types=(
            pltpu.VMEM((_NCH, _CH), jnp.int32),
            pltpu.VMEM((_CH, _D), jnp.float32),
            pltpu.VMEM((_CH, _D), jnp.float32),
            pltpu.VMEM((_CH, _D), jnp.float32),
            pltpu.VMEM((_CH, _D), jnp.float32),
            pltpu.SemaphoreType.DMA,
            pltpu.SemaphoreType.DMA,
            pltpu.SemaphoreType.DMA,
            pltpu.SemaphoreType.DMA,
        ),
    )
    return fn(emb, emb2, idx3)


def _mm(x, y):
    return jnp.dot(x.astype(jnp.bfloat16), y.astype(jnp.bfloat16),
                   preferred_element_type=jnp.float32)


def _softmax_lanes(e):
    m = jnp.max(e, axis=1, keepdims=True)
    p = jnp.exp(e - m)
    return p / jnp.sum(p, axis=1, keepdims=True)


def _pad_stack(flat, ncols):
    z = jnp.zeros((_S - _L, ncols), jnp.float32)
    pieces = []
    for i in range(_G):
        pieces.append(flat[i * _L:(i + 1) * _L])
        pieces.append(z)
    return jnp.concatenate(pieces, axis=0)


def _one_stack(xf, hts, bd, e64, wa1, wa2n, wa2e, c0):
    zl = jnp.zeros((_E, _S - _L), jnp.float32)
    zr = jnp.zeros((_S - _E, _S), jnp.float32)
    pieces = []
    for h in hts:
        pieces.append(jnp.concatenate([h, zl], axis=1))
        pieces.append(zr)
    mask1 = jnp.concatenate(pieces, axis=0) > 0.0
    xp = _pad_stack(xf, _D)

    s1 = _mm(xp, wa1) + c0
    s1 = jnp.where(s1 >= 0, s1, 0.2 * s1)
    e1 = _mm(bd, s1 * e64)
    att1 = _softmax_lanes(jnp.where(mask1, e1, _NEG))
    a1 = jnp.concatenate([att1] * _G, axis=1) * bd
    edge = _mm(a1, xp)
    s2n = _mm(xp, wa2n)
    s2e = _mm(edge, wa2e)
    s2n_l = _mm(bd, s2n * e64)
    e2 = s2n_l + s2e
    e2 = jnp.where(e2 >= 0, e2, 0.2 * e2)
    p2 = jnp.where(mask1, jnp.exp(e2), 0.0)
    den = _mm(bd, p2)
    att2 = jnp.where(den > 0, p2 / den, 1.0 / _E)
    a2m = jnp.concatenate([att2] * _G, axis=1) * bd
    node = lax.dot_general(a2m.astype(jnp.bfloat16), edge.astype(jnp.bfloat16),
                           (((0,), (0,)), ((), ())),
                           preferred_element_type=jnp.float32)
    return node + xp


def _attn_body(xf_ref, n2_ref, ht_ref, bd_ref, e64_ref,
               wa1_ref, wa2n_ref, wa2e_ref, c0_ref, o_ref, o2_ref):
    wa1 = wa1_ref[...]
    wa2n = wa2n_ref[...]
    wa2e = wa2e_ref[...]
    c0 = c0_ref[0, 0]
    bd = bd_ref[...]
    e64 = e64_ref[...]

    for k in range(_NSTACK):
        xf = xf_ref[pl.ds(k * _GL, _GL), :]
        hts = [ht_ref[_G * k + i] for i in range(_G)]
        res = _one_stack(xf, hts, bd, e64, wa1, wa2n, wa2e, c0)
        for i in range(_G):
            o_ref[_G * k + i] = res[i * _S:i * _S + _L]
    for i in range(_NSTACK * _G):
        o2_ref[i] = n2_ref[pl.ds(i * _L, _L), :]


def _tc_attention(nodes_flat, nodes2_flat, HT, bd, e64, wa1, wa2n, wa2e, c0):
    rows = _NSTACK * _GL
    nsess = _NSTACK * _G
    grid = (_B // nsess,)
    return pl.pallas_call(
        _attn_body,
        grid=grid,
        in_specs=[
            pl.BlockSpec((rows, _D), lambda i: (i, 0)),
            pl.BlockSpec((rows, _D), lambda i: (i, 0)),
            pl.BlockSpec((nsess, _E, _L), lambda i: (i, 0, 0)),
            pl.BlockSpec((_R, _R), lambda i: (0, 0)),
            pl.BlockSpec((_R, _S), lambda i: (0, 0)),
            pl.BlockSpec((_D, 1), lambda i: (0, 0)),
            pl.BlockSpec((_D, 1), lambda i: (0, 0)),
            pl.BlockSpec((_D, 1), lambda i: (0, 0)),
            pl.BlockSpec((1, 1), lambda i: (0, 0)),
        ],
        out_specs=[
            pl.BlockSpec((nsess, _L, _D), lambda i: (i, 0, 0)),
            pl.BlockSpec((nsess, _L, _D), lambda i: (i, 0, 0)),
        ],
        out_shape=[
            jax.ShapeDtypeStruct((_B, _L, _D), jnp.float32),
            jax.ShapeDtypeStruct((_B, _L, _D), jnp.float32),
        ],
    )(nodes_flat, nodes2_flat, HT, bd, e64, wa1, wa2n, wa2e, c0)


_BD = (np.arange(_R)[:, None] // _S == np.arange(_R)[None, :] // _S).astype(
    np.float32)
_E64 = (np.arange(_R)[:, None] % _S == np.arange(_S)[None, :]).astype(
    np.float32)


def kernel(inputs, HT, G, EG, emb, emb2, w2, w3, a, a2, ctx):
    idx3 = inputs.reshape(_NW, _NCH, _CH).astype(jnp.int32)
    nodes_flat, nodes2_flat = _sc_gather2(emb, emb2, idx3)
    wa1 = w2 @ a[_D:, :]
    wa2n = w2 @ a2[:_D, :]
    wa2e = w3 @ a2[_D:, :]
    c0 = (ctx @ a[:_D, :]).reshape(1, 1)
    out, nodes2 = _tc_attention(nodes_flat, nodes2_flat, HT,
                                _BD, _E64, wa1, wa2n, wa2e, c0)
    return (out, out, nodes2)


# speedup vs baseline: 1.0476x; 1.0476x over previous
import numpy as np

import jax
import jax.numpy as jnp
from jax import lax
from jax.experimental import pallas as pl
from jax.experimental.pallas import tpu as pltpu
from jax.experimental.pallas import tpu_sc as plsc

_B = 1024
_L = 50
_E = 50
_D = 128
_BL = _B * _L

_NC = 2
_NS = 16
_NW = _NC * _NS
_PER_W = _BL // _NW
_CH = 80
_NCH = _PER_W // _CH

_G = 4
_S = 64
_R = _G * _S
_GL = _G * _L
_NSTACK = 8
_NEG = -9e15


def _sc_gather_body(emb_hbm, emb2_hbm, idx_hbm, out1_hbm, out2_hbm,
                    idx_v, b1a, b2a, b1b, b2b, s1a, s2a, s1b, s2b):
    wid = lax.axis_index("s") * _NC + lax.axis_index("c")
    pltpu.sync_copy(idx_hbm.at[wid], idx_v)
    base = wid * _PER_W

    def pair(c, carry):
        ca = 2 * c
        off_a = base + ca * _CH
        off_b = off_a + _CH
        cp1a = pltpu.async_copy(emb_hbm.at[idx_v.at[ca]], b1a, s1a)
        cp2a = pltpu.async_copy(emb2_hbm.at[idx_v.at[ca]], b2a, s2a)
        cp1b = pltpu.async_copy(emb_hbm.at[idx_v.at[ca + 1]], b1b, s1b)
        cp2b = pltpu.async_copy(emb2_hbm.at[idx_v.at[ca + 1]], b2b, s2b)
        cp1a.wait()
        pltpu.sync_copy(b1a, out1_hbm.at[pl.ds(off_a, _CH)])
        cp2a.wait()
        pltpu.sync_copy(b2a, out2_hbm.at[pl.ds(off_a, _CH)])
        cp1b.wait()
        pltpu.sync_copy(b1b, out1_hbm.at[pl.ds(off_b, _CH)])
        cp2b.wait()
        pltpu.sync_copy(b2b, out2_hbm.at[pl.ds(off_b, _CH)])
        return carry

    lax.fori_loop(0, _NCH // 2, pair, 0)


def _sc_gather2(emb, emb2, idx3):
    mesh = plsc.VectorSubcoreMesh(core_axis_name="c", subcore_axis_name="s")
    fn = pl.kernel(
        _sc_gather_body,
        out_type=(
            jax.ShapeDtypeStruct((_BL, _D), jnp.float32),
            jax.ShapeDtypeStruct((_BL, _D), jnp.float32),
        ),
        mesh=mesh,
        scratch_types=(
            pltpu.VMEM((_NCH, _CH), jnp.int32),
            pltpu.VMEM((_CH, _D), jnp.float32),
            pltpu.VMEM((_CH, _D), jnp.float32),
            pltpu.VMEM((_CH, _D), jnp.float32),
            pltpu.VMEM((_CH, _D), jnp.float32),
            pltpu.SemaphoreType.DMA,
            pltpu.SemaphoreType.DMA,
            pltpu.SemaphoreType.DMA,
            pltpu.SemaphoreType.DMA,
        ),
    )
    return fn(emb, emb2, idx3)


def _mm(x, y):
    return jnp.dot(x.astype(jnp.bfloat16), y.astype(jnp.bfloat16),
                   preferred_element_type=jnp.float32)


def _softmax_lanes(e):
    m = jnp.max(e, axis=1, keepdims=True)
    p = jnp.exp(e - m)
    return p / jnp.sum(p, axis=1, keepdims=True)


def _pad_stack(flat, ncols):
    z = jnp.zeros((_S - _L, ncols), jnp.float32)
    pieces = []
    for i in range(_G):
        pieces.append(flat[i * _L:(i + 1) * _L])
        pieces.append(z)
    return jnp.concatenate(pieces, axis=0)


def _one_stack(xf, hts, bd, e64, wa1, wa2n, wa2e, c0):
    zl = jnp.zeros((_E, _S - _L), jnp.float32)
    zr = jnp.zeros((_S - _E, _S), jnp.float32)
    pieces = []
    for h in hts:
        pieces.append(jnp.concatenate([h, zl], axis=1))
        pieces.append(zr)
    mask1 = jnp.concatenate(pieces, axis=0) > 0.0
    xp = _pad_stack(xf, _D)

    s1 = _mm(xp, wa1) + c0
    s1 = jnp.where(s1 >= 0, s1, 0.2 * s1)
    e1 = _mm(bd, s1 * e64)
    p1 = jnp.where(mask1, jnp.exp(e1), 0.0)
    den1 = jnp.sum(p1, axis=1, keepdims=True)
    att1 = p1 / (den1 + (den1 <= 0))
    a1 = jnp.concatenate([att1] * _G, axis=1) * bd
    edge = _mm(a1, xp)
    s2n = _mm(xp, wa2n)
    s2e = _mm(edge, wa2e)
    s2n_l = _mm(bd, s2n * e64)
    e2 = s2n_l + s2e
    e2 = jnp.where(e2 >= 0, e2, 0.2 * e2)
    p2 = jnp.where(mask1, jnp.exp(e2), 0.0)
    den = _mm(bd, p2)
    att2 = jnp.where(den > 0, p2 / den, 1.0 / _E)
    a2m = jnp.concatenate([att2] * _G, axis=1) * bd
    node = lax.dot_general(a2m.astype(jnp.bfloat16), edge.astype(jnp.bfloat16),
                           (((0,), (0,)), ((), ())),
                           preferred_element_type=jnp.float32)
    return node + xp


def _attn_body(xf_ref, n2_ref, ht_ref, bd_ref, e64_ref,
               wa1_ref, wa2n_ref, wa2e_ref, c0_ref, o_ref, o2_ref):
    wa1 = wa1_ref[...]
    wa2n = wa2n_ref[...]
    wa2e = wa2e_ref[...]
    c0 = c0_ref[0, 0]
    bd = bd_ref[...]
    e64 = e64_ref[...]

    for k in range(_NSTACK):
        xf = xf_ref[pl.ds(k * _GL, _GL), :]
        hts = [ht_ref[_G * k + i] for i in range(_G)]
        res = _one_stack(xf, hts, bd, e64, wa1, wa2n, wa2e, c0)
        for i in range(_G):
            o_ref[_G * k + i] = res[i * _S:i * _S + _L]
    for i in range(_NSTACK * _G):
        o2_ref[i] = n2_ref[pl.ds(i * _L, _L), :]


def _tc_attention(nodes_flat, nodes2_flat, HT, bd, e64, wa1, wa2n, wa2e, c0):
    rows = _NSTACK * _GL
    nsess = _NSTACK * _G
    grid = (_B // nsess,)
    return pl.pallas_call(
        _attn_body,
        grid=grid,
        in_specs=[
            pl.BlockSpec((rows, _D), lambda i: (i, 0)),
            pl.BlockSpec((rows, _D), lambda i: (i, 0)),
            pl.BlockSpec((nsess, _E, _L), lambda i: (i, 0, 0)),
            pl.BlockSpec((_R, _R), lambda i: (0, 0)),
            pl.BlockSpec((_R, _S), lambda i: (0, 0)),
            pl.BlockSpec((_D, 1), lambda i: (0, 0)),
            pl.BlockSpec((_D, 1), lambda i: (0, 0)),
            pl.BlockSpec((_D, 1), lambda i: (0, 0)),
            pl.BlockSpec((1, 1), lambda i: (0, 0)),
        ],
        out_specs=[
            pl.BlockSpec((nsess, _L, _D), lambda i: (i, 0, 0)),
            pl.BlockSpec((nsess, _L, _D), lambda i: (i, 0, 0)),
        ],
        out_shape=[
            jax.ShapeDtypeStruct((_B, _L, _D), jnp.float32),
            jax.ShapeDtypeStruct((_B, _L, _D), jnp.float32),
        ],
    )(nodes_flat, nodes2_flat, HT, bd, e64, wa1, wa2n, wa2e, c0)


_BD = (np.arange(_R)[:, None] // _S == np.arange(_R)[None, :] // _S).astype(
    np.float32)
_E64 = (np.arange(_R)[:, None] % _S == np.arange(_S)[None, :]).astype(
    np.float32)


def kernel(inputs, HT, G, EG, emb, emb2, w2, w3, a, a2, ctx):
    idx3 = inputs.reshape(_NW, _NCH, _CH).astype(jnp.int32)
    nodes_flat, nodes2_flat = _sc_gather2(emb, emb2, idx3)
    wa1 = w2 @ a[_D:, :]
    wa2n = w2 @ a2[:_D, :]
    wa2e = w3 @ a2[_D:, :]
    c0 = (ctx @ a[:_D, :]).reshape(1, 1)
    out, nodes2 = _tc_attention(nodes_flat, nodes2_flat, HT,
                                _BD, _E64, wa1, wa2n, wa2e, c0)
    return (out, out, nodes2)


# speedup vs baseline: 1.0619x; 1.0136x over previous
import numpy as np

import jax
import jax.numpy as jnp
from jax import lax
from jax.experimental import pallas as pl
from jax.experimental.pallas import tpu as pltpu
from jax.experimental.pallas import tpu_sc as plsc

_B = 1024
_L = 50
_E = 50
_D = 128
_BL = _B * _L

_NC = 2
_NS = 16
_NW = _NC * _NS
_PER_W = _BL // _NW
_CH = 80
_NCH = _PER_W // _CH

_G = 4
_S = 64
_R = _G * _S
_GL = _G * _L
_NSTACK = 16
_NEG = -9e15


def _sc_gather_body(emb_hbm, emb2_hbm, idx_hbm, out1_hbm, out2_hbm,
                    idx_v, b1a, b2a, b1b, b2b, s1a, s2a, s1b, s2b):
    wid = lax.axis_index("s") * _NC + lax.axis_index("c")
    pltpu.sync_copy(idx_hbm.at[wid], idx_v)
    base = wid * _PER_W

    def pair(c, carry):
        ca = 2 * c
        off_a = base + ca * _CH
        off_b = off_a + _CH
        cp1a = pltpu.async_copy(emb_hbm.at[idx_v.at[ca]], b1a, s1a)
        cp2a = pltpu.async_copy(emb2_hbm.at[idx_v.at[ca]], b2a, s2a)
        cp1b = pltpu.async_copy(emb_hbm.at[idx_v.at[ca + 1]], b1b, s1b)
        cp2b = pltpu.async_copy(emb2_hbm.at[idx_v.at[ca + 1]], b2b, s2b)
        cp1a.wait()
        pltpu.sync_copy(b1a, out1_hbm.at[pl.ds(off_a, _CH)])
        cp2a.wait()
        pltpu.sync_copy(b2a, out2_hbm.at[pl.ds(off_a, _CH)])
        cp1b.wait()
        pltpu.sync_copy(b1b, out1_hbm.at[pl.ds(off_b, _CH)])
        cp2b.wait()
        pltpu.sync_copy(b2b, out2_hbm.at[pl.ds(off_b, _CH)])
        return carry

    lax.fori_loop(0, _NCH // 2, pair, 0)


def _sc_gather2(emb, emb2, idx3):
    mesh = plsc.VectorSubcoreMesh(core_axis_name="c", subcore_axis_name="s")
    fn = pl.kernel(
        _sc_gather_body,
        out_type=(
            jax.ShapeDtypeStruct((_BL, _D), jnp.float32),
            jax.ShapeDtypeStruct((_BL, _D), jnp.float32),
        ),
        mesh=mesh,
        scratch_types=(
            pltpu.VMEM((_NCH, _CH), jnp.int32),
            pltpu.VMEM((_CH, _D), jnp.float32),
            pltpu.VMEM((_CH, _D), jnp.float32),
            pltpu.VMEM((_CH, _D), jnp.float32),
            pltpu.VMEM((_CH, _D), jnp.float32),
            pltpu.SemaphoreType.DMA,
            pltpu.SemaphoreType.DMA,
            pltpu.SemaphoreType.DMA,
            pltpu.SemaphoreType.DMA,
        ),
    )
    return fn(emb, emb2, idx3)


def _mm(x, y):
    return jnp.dot(x.astype(jnp.bfloat16), y.astype(jnp.bfloat16),
                   preferred_element_type=jnp.float32)


def _softmax_lanes(e):
    m = jnp.max(e, axis=1, keepdims=True)
    p = jnp.exp(e - m)
    return p / jnp.sum(p, axis=1, keepdims=True)


def _pad_stack(flat, ncols):
    z = jnp.zeros((_S - _L, ncols), jnp.float32)
    pieces = []
    for i in range(_G):
        pieces.append(flat[i * _L:(i + 1) * _L])
        pieces.append(z)
    return jnp.concatenate(pieces, axis=0)


def _one_stack(xf, hts, bd, e64, wa1, wa2n, wa2e, c0):
    zl = jnp.zeros((_E, _S - _L), jnp.float32)
    zr = jnp.zeros((_S - _E, _S), jnp.float32)
    pieces = []
    for h in hts:
        pieces.append(jnp.concatenate([h, zl], axis=1))
        pieces.append(zr)
    mask1 = jnp.concatenate(pieces, axis=0) > 0.0
    xp = _pad_stack(xf, _D)

    s1 = _mm(xp, wa1) + c0
    s1 = jnp.where(s1 >= 0, s1, 0.2 * s1)
    e1 = _mm(bd, s1 * e64)
    p1 = jnp.where(mask1, jnp.exp(e1), 0.0)
    den1 = jnp.sum(p1, axis=1, keepdims=True)
    att1 = p1 / (den1 + (den1 <= 0))
    a1 = jnp.concatenate([att1] * _G, axis=1) * bd
    edge = _mm(a1, xp)
    s2n = _mm(xp, wa2n)
    s2e = _mm(edge, wa2e)
    s2n_l = _mm(bd, s2n * e64)
    e2 = s2n_l + s2e
    e2 = jnp.where(e2 >= 0, e2, 0.2 * e2)
    p2 = jnp.where(mask1, jnp.exp(e2), 0.0)
    den = _mm(bd, p2)
    att2 = jnp.where(den > 0, p2 / den, 1.0 / _E)
    a2m = jnp.concatenate([att2] * _G, axis=1) * bd
    node = lax.dot_general(a2m.astype(jnp.bfloat16), edge.astype(jnp.bfloat16),
                           (((0,), (0,)), ((), ())),
                           preferred_element_type=jnp.float32)
    return node + xp


def _attn_body(xf_ref, n2_ref, ht_ref, bd_ref, e64_ref,
               wa1_ref, wa2n_ref, wa2e_ref, c0_ref, o_ref, o2_ref):
    wa1 = wa1_ref[...]
    wa2n = wa2n_ref[...]
    wa2e = wa2e_ref[...]
    c0 = c0_ref[0, 0]
    bd = bd_ref[...]
    e64 = e64_ref[...]

    for k in range(_NSTACK):
        xf = xf_ref[pl.ds(k * _GL, _GL), :]
        hts = [ht_ref[_G * k + i] for i in range(_G)]
        res = _one_stack(xf, hts, bd, e64, wa1, wa2n, wa2e, c0)
        for i in range(_G):
            o_ref[_G * k + i] = res[i * _S:i * _S + _L]
    for i in range(_NSTACK * _G):
        o2_ref[i] = n2_ref[pl.ds(i * _L, _L), :]


def _tc_attention(nodes_flat, nodes2_flat, HT, bd, e64, wa1, wa2n, wa2e, c0):
    rows = _NSTACK * _GL
    nsess = _NSTACK * _G
    grid = (_B // nsess,)
    return pl.pallas_call(
        _attn_body,
        grid=grid,
        in_specs=[
            pl.BlockSpec((rows, _D), lambda i: (i, 0)),
            pl.BlockSpec((rows, _D), lambda i: (i, 0)),
            pl.BlockSpec((nsess, _E, _L), lambda i: (i, 0, 0)),
            pl.BlockSpec((_R, _R), lambda i: (0, 0)),
            pl.BlockSpec((_R, _S), lambda i: (0, 0)),
            pl.BlockSpec((_D, 1), lambda i: (0, 0)),
            pl.BlockSpec((_D, 1), lambda i: (0, 0)),
            pl.BlockSpec((_D, 1), lambda i: (0, 0)),
            pl.BlockSpec((1, 1), lambda i: (0, 0)),
        ],
        out_specs=[
            pl.BlockSpec((nsess, _L, _D), lambda i: (i, 0, 0)),
            pl.BlockSpec((nsess, _L, _D), lambda i: (i, 0, 0)),
        ],
        out_shape=[
            jax.ShapeDtypeStruct((_B, _L, _D), jnp.float32),
            jax.ShapeDtypeStruct((_B, _L, _D), jnp.float32),
        ],
    )(nodes_flat, nodes2_flat, HT, bd, e64, wa1, wa2n, wa2e, c0)


_BD = (np.arange(_R)[:, None] // _S == np.arange(_R)[None, :] // _S).astype(
    np.float32)
_E64 = (np.arange(_R)[:, None] % _S == np.arange(_S)[None, :]).astype(
    np.float32)


def kernel(inputs, HT, G, EG, emb, emb2, w2, w3, a, a2, ctx):
    idx3 = inputs.reshape(_NW, _NCH, _CH).astype(jnp.int32)
    nodes_flat, nodes2_flat = _sc_gather2(emb, emb2, idx3)
    wa1 = w2 @ a[_D:, :]
    wa2n = w2 @ a2[:_D, :]
    wa2e = w3 @ a2[_D:, :]
    c0 = (ctx @ a[:_D, :]).reshape(1, 1)
    out, nodes2 = _tc_attention(nodes_flat, nodes2_flat, HT,
                                _BD, _E64, wa1, wa2n, wa2e, c0)
    return (out, out, nodes2)


# speedup vs baseline: 1.0630x; 1.0011x over previous
import numpy as np

import jax
import jax.numpy as jnp
from jax import lax
from jax.experimental import pallas as pl
from jax.experimental.pallas import tpu as pltpu
from jax.experimental.pallas import tpu_sc as plsc

_B = 1024
_L = 50
_E = 50
_D = 128
_BL = _B * _L

_NC = 2
_NS = 16
_NW = _NC * _NS
_PER_W = _BL // _NW
_CH = 80
_NCH = _PER_W // _CH

_G = 4
_S = 64
_R = _G * _S
_GL = _G * _L
_NSTACK = 16


def _sc_gather_body(emb_hbm, emb2_hbm, idx_hbm, out1_hbm, out2_hbm,
                    idx_v, b1a, b2a, b1b, b2b, s1a, s2a, s1b, s2b):
    wid = lax.axis_index("s") * _NC + lax.axis_index("c")
    pltpu.sync_copy(idx_hbm.at[wid], idx_v)
    base = wid * _PER_W

    def pair(c, carry):
        ca = 2 * c
        off_a = base + ca * _CH
        off_b = off_a + _CH
        cp1a = pltpu.async_copy(emb_hbm.at[idx_v.at[ca]], b1a, s1a)
        cp2a = pltpu.async_copy(emb2_hbm.at[idx_v.at[ca]], b2a, s2a)
        cp1b = pltpu.async_copy(emb_hbm.at[idx_v.at[ca + 1]], b1b, s1b)
        cp2b = pltpu.async_copy(emb2_hbm.at[idx_v.at[ca + 1]], b2b, s2b)
        cp1a.wait()
        pltpu.sync_copy(b1a, out1_hbm.at[pl.ds(off_a, _CH)])
        cp2a.wait()
        pltpu.sync_copy(b2a, out2_hbm.at[pl.ds(off_a, _CH)])
        cp1b.wait()
        pltpu.sync_copy(b1b, out1_hbm.at[pl.ds(off_b, _CH)])
        cp2b.wait()
        pltpu.sync_copy(b2b, out2_hbm.at[pl.ds(off_b, _CH)])
        return carry

    lax.fori_loop(0, _NCH // 2, pair, 0)


def _sc_gather2(emb, emb2, idx3):
    mesh = plsc.VectorSubcoreMesh(core_axis_name="c", subcore_axis_name="s")
    fn = pl.kernel(
        _sc_gather_body,
        out_type=(
            jax.ShapeDtypeStruct((_BL, _D), jnp.float32),
            jax.ShapeDtypeStruct((_BL, _D), jnp.float32),
        ),
        mesh=mesh,
        scratch_types=(
            pltpu.VMEM((_NCH, _CH), jnp.int32),
            pltpu.VMEM((_CH, _D), jnp.float32),
            pltpu.VMEM((_CH, _D), jnp.float32),
            pltpu.VMEM((_CH, _D), jnp.float32),
            pltpu.VMEM((_CH, _D), jnp.float32),
            pltpu.SemaphoreType.DMA,
            pltpu.SemaphoreType.DMA,
            pltpu.SemaphoreType.DMA,
            pltpu.SemaphoreType.DMA,
        ),
    )
    return fn(emb, emb2, idx3)


def _mm(x, y):
    return jnp.dot(x.astype(jnp.bfloat16), y.astype(jnp.bfloat16),
                   preferred_element_type=jnp.float32)


def _pad_stack(flat, ncols):
    z = jnp.zeros((_S - _L, ncols), jnp.float32)
    pieces = []
    for i in range(_G):
        pieces.append(flat[i * _L:(i + 1) * _L])
        pieces.append(z)
    return jnp.concatenate(pieces, axis=0)


def _one_stack(xf, hts, bd, e64, wa1, wa2n, wa2e, c0):
    zl = jnp.zeros((_E, _S - _L), jnp.float32)
    zr = jnp.zeros((_S - _E, _S), jnp.float32)
    pieces = []
    for h in hts:
        pieces.append(jnp.concatenate([h, zl], axis=1))
        pieces.append(zr)
    mask1 = jnp.concatenate(pieces, axis=0) > 0.0
    xp = _pad_stack(xf, _D)

    s1 = _mm(xp, wa1) + c0
    s1 = jnp.where(s1 >= 0, s1, 0.2 * s1)
    e1 = _mm(bd, s1 * e64)
    p1 = jnp.where(mask1, jnp.exp(e1), 0.0)
    den1 = jnp.sum(p1, axis=1, keepdims=True)
    att1 = p1 / (den1 + (den1 <= 0))
    a1 = jnp.concatenate([att1] * _G, axis=1) * bd
    edge = _mm(a1, xp)
    s2n = _mm(xp, wa2n)
    s2e = _mm(edge, wa2e)
    s2n_l = _mm(bd, s2n * e64)
    e2 = s2n_l + s2e
    e2 = jnp.where(e2 >= 0, e2, 0.2 * e2)
    p2 = jnp.where(mask1, jnp.exp(e2), 0.0)
    den = _mm(bd, p2)
    att2 = jnp.where(den > 0, p2 / den, 1.0 / _E)
    a2m = jnp.concatenate([att2] * _G, axis=1) * bd
    node = lax.dot_general(a2m.astype(jnp.bfloat16), edge.astype(jnp.bfloat16),
                           (((0,), (0,)), ((), ())),
                           preferred_element_type=jnp.float32)
    return node + xp


def _attn_body(xf_ref, n2_ref, ht_ref, bd_ref, e64_ref,
               wa1_ref, wa2n_ref, wa2e_ref, c0_ref, o_ref, o2_ref):
    wa1 = wa1_ref[...]
    wa2n = wa2n_ref[...]
    wa2e = wa2e_ref[...]
    c0 = c0_ref[0, 0]
    bd = bd_ref[...]
    e64 = e64_ref[...]

    for k in range(_NSTACK):
        xf = xf_ref[pl.ds(k * _GL, _GL), :]
        hts = [ht_ref[_G * k + i] for i in range(_G)]
        res = _one_stack(xf, hts, bd, e64, wa1, wa2n, wa2e, c0)
        for i in range(_G):
            o_ref[_G * k + i] = res[i * _S:i * _S + _L]
    for i in range(_NSTACK * _G):
        o2_ref[i] = n2_ref[pl.ds(i * _L, _L), :]


def _tc_attention(nodes_flat, nodes2_flat, HT, bd, e64, wa1, wa2n, wa2e, c0):
    rows = _NSTACK * _GL
    nsess = _NSTACK * _G
    grid = (_B // nsess,)
    return pl.pallas_call(
        _attn_body,
        grid=grid,
        in_specs=[
            pl.BlockSpec((rows, _D), lambda i: (i, 0)),
            pl.BlockSpec((rows, _D), lambda i: (i, 0)),
            pl.BlockSpec((nsess, _E, _L), lambda i: (i, 0, 0)),
            pl.BlockSpec((_R, _R), lambda i: (0, 0)),
            pl.BlockSpec((_R, _S), lambda i: (0, 0)),
            pl.BlockSpec((_D, 1), lambda i: (0, 0)),
            pl.BlockSpec((_D, 1), lambda i: (0, 0)),
            pl.BlockSpec((_D, 1), lambda i: (0, 0)),
            pl.BlockSpec((1, 1), lambda i: (0, 0)),
        ],
        out_specs=[
            pl.BlockSpec((nsess, _L, _D), lambda i: (i, 0, 0)),
            pl.BlockSpec((nsess, _L, _D), lambda i: (i, 0, 0)),
        ],
        out_shape=[
            jax.ShapeDtypeStruct((_B, _L, _D), jnp.float32),
            jax.ShapeDtypeStruct((_B, _L, _D), jnp.float32),
        ],
    )(nodes_flat, nodes2_flat, HT, bd, e64, wa1, wa2n, wa2e, c0)


_BD = (np.arange(_R)[:, None] // _S == np.arange(_R)[None, :] // _S).astype(
    np.float32)
_E64 = (np.arange(_R)[:, None] % _S == np.arange(_S)[None, :]).astype(
    np.float32)


def kernel(inputs, HT, G, EG, emb, emb2, w2, w3, a, a2, ctx):
    idx3 = inputs.reshape(_NW, _NCH, _CH).astype(jnp.int32)
    nodes_flat, nodes2_flat = _sc_gather2(emb, emb2, idx3)
    wa1 = w2 @ a[_D:, :]
    wa2n = w2 @ a2[:_D, :]
    wa2e = w3 @ a2[_D:, :]
    c0 = (ctx @ a[:_D, :]).reshape(1, 1)
    out, nodes2 = _tc_attention(nodes_flat, nodes2_flat, HT,
                                _BD, _E64, wa1, wa2n, wa2e, c0)
    return (out, out, nodes2)
